# baseline (speedup 1.0000x reference)
import functools

import jax
import jax.numpy as jnp
from jax import lax
from jax.experimental import pallas as pl
from jax.experimental.pallas import tpu as pltpu
from jax.experimental.pallas import tpu_sc as plsc

BATCH = 128
SEQ = 200
D = 2048
TOTAL = BATCH * SEQ
NC = 2
NS = 16
NW = NC * NS
NSLICE = 4
CSLICE = D // NSLICE
RANGES = NW // NSLICE
R_PER_RANGE = TOTAL // RANGES
CHUNK = 16
NCH = R_PER_RANGE // CHUNK


def _body(idx_hbm, table_hbm, out_hbm, idx_v, tbl_v, obuf, wsem, tsem):
    wid = lax.axis_index("s") * NC + lax.axis_index("c")
    rid = wid // NSLICE
    cs = (wid % NSLICE) * CSLICE
    row0 = rid * R_PER_RANGE

    pltpu.make_async_copy(
        table_hbm.at[:, pl.ds(cs, CSLICE)], tbl_v, tsem).start()
    pltpu.sync_copy(idx_hbm.at[pl.ds(row0, R_PER_RANGE)], idx_v)
    pltpu.make_async_copy(
        table_hbm.at[:, pl.ds(cs, CSLICE)], tbl_v, tsem).wait()

    def w_copy(j, b):
        return pltpu.make_async_copy(
            obuf.at[b],
            out_hbm.at[pl.ds(row0 + j * CHUNK, CHUNK), pl.ds(cs, CSLICE)],
            wsem)

    def expand(j, b):
        iv = idx_v[pl.ds(j * CHUNK, CHUNK)]
        ixs = [iv[r] for r in range(CHUNK)]

        @plsc.parallel_loop(0, CSLICE // 16, 1, unroll=16)
        def _(k):
            for r in range(CHUNK):
                obuf[b, r, pl.ds(k * 16, 16)] = tbl_v[ixs[r], pl.ds(k * 16, 16)]

    expand(0, 0)
    w_copy(0, 0).start()
    expand(1, 1)
    w_copy(1, 1).start()

    def pair(p, carry):
        j0 = 2 * p + 2
        for t in range(2):
            j = j0 + t
            w_copy(j - 2, t).wait()
            expand(j, t)
            w_copy(j, t).start()
        return carry

    lax.fori_loop(0, (NCH - 2) // 2, pair, 0)
    w_copy(NCH - 2, 0).wait()
    w_copy(NCH - 1, 1).wait()


_gather = functools.partial(
    pl.kernel,
    mesh=plsc.VectorSubcoreMesh(core_axis_name="c", subcore_axis_name="s"),
    out_type=jax.ShapeDtypeStruct((TOTAL, D), jnp.float32),
    scratch_types=[
        pltpu.VMEM((R_PER_RANGE,), jnp.int32),
        pltpu.VMEM((SEQ, CSLICE), jnp.float32),
        pltpu.VMEM((2, CHUNK, CSLICE), jnp.float32),
        pltpu.SemaphoreType.DMA,
        pltpu.SemaphoreType.DMA,
    ],
)(_body)


def kernel(indices, weight):
    idx = indices.astype(jnp.int32).reshape(TOTAL)
    out = _gather(idx, weight)
    return out.reshape(BATCH, SEQ, D)


# speedup vs baseline: 1.1735x; 1.1735x over previous
import functools

import jax
import jax.numpy as jnp
from jax import lax
from jax.experimental import pallas as pl
from jax.experimental.pallas import tpu as pltpu
from jax.experimental.pallas import tpu_sc as plsc

BATCH = 128
SEQ = 200
D = 2048
TOTAL = BATCH * SEQ
NC = 2
NS = 16
NW = NC * NS
NSLICE = 4
CSLICE = D // NSLICE
RANGES = NW // NSLICE
R_PER_RANGE = TOTAL // RANGES
CHUNK = 16
NCH = R_PER_RANGE // CHUNK


def _body(idx_hbm, table_hbm, out_hbm, idx_v, tbl_v, obuf, wsem, tsem):
    wid = lax.axis_index("s") * NC + lax.axis_index("c")
    rid = wid // NSLICE
    cs = (wid % NSLICE) * CSLICE
    row0 = rid * R_PER_RANGE

    pltpu.make_async_copy(
        table_hbm.at[:, pl.ds(cs, CSLICE)], tbl_v, tsem).start()
    pltpu.sync_copy(idx_hbm.at[pl.ds(row0, R_PER_RANGE)], idx_v)
    pltpu.make_async_copy(
        table_hbm.at[:, pl.ds(cs, CSLICE)], tbl_v, tsem).wait()

    def w_copy(j, b):
        return pltpu.make_async_copy(
            obuf.at[b],
            out_hbm.at[pl.ds(row0 + j * CHUNK, CHUNK), pl.ds(cs, CSLICE)],
            wsem)

    def expand(j, b):
        iv = idx_v[pl.ds(j * CHUNK, CHUNK)]
        ixs = [iv[r] for r in range(CHUNK)]

        @plsc.parallel_loop(0, CSLICE // 16, 1, unroll=8)
        def _(k):
            for r in range(CHUNK):
                obuf[b, r, pl.ds(k * 16, 16)] = tbl_v[ixs[r], pl.ds(k * 16, 16)]

    for j in range(3):
        expand(j, j)
        w_copy(j, j).start()

    def group(gi, carry):
        j0 = 3 * gi + 3
        for t in range(3):
            j = j0 + t
            w_copy(j - 3, t).wait()
            expand(j, t)
            w_copy(j, t).start()
        return carry

    lax.fori_loop(0, (NCH - 5) // 3, group, 0)

    for j in (NCH - 2, NCH - 1):
        w_copy(j - 3, j % 3).wait()
        expand(j, j % 3)
        w_copy(j, j % 3).start()
    for j in (NCH - 3, NCH - 2, NCH - 1):
        w_copy(j, j % 3).wait()


_gather = functools.partial(
    pl.kernel,
    mesh=plsc.VectorSubcoreMesh(core_axis_name="c", subcore_axis_name="s"),
    out_type=jax.ShapeDtypeStruct((TOTAL, D), jnp.float32),
    scratch_types=[
        pltpu.VMEM((R_PER_RANGE,), jnp.int32),
        pltpu.VMEM((SEQ, CSLICE), jnp.float32),
        pltpu.VMEM((3, CHUNK, CSLICE), jnp.float32),
        pltpu.SemaphoreType.DMA,
        pltpu.SemaphoreType.DMA,
    ],
)(_body)


def kernel(indices, weight):
    idx = indices.astype(jnp.int32).reshape(TOTAL)
    out = _gather(idx, weight)
    return out.reshape(BATCH, SEQ, D)
